# baseline (speedup 1.0000x reference)
import math
import functools

import jax
import jax.numpy as jnp
from jax import lax
from jax.experimental import pallas as pl
from jax.experimental.pallas import tpu as pltpu
from jax.experimental.pallas import tpu_sc as plsc

_VOCAB = 256
_D = 2048
_SCALE = math.sqrt(_D)

_NC = 2
_NS = 16
_NW = _NC * _NS

_C = 16
_RB = 256

_SC_FRAC_NUM = 1
_SC_FRAC_DEN = 2


def _prescale_body(t_ref, of_ref, ob_ref):
    row = lax.broadcasted_iota(jnp.int32, (_VOCAB, _D), 0)
    v = jnp.where(row == 0, 0.0, t_ref[...] * _SCALE)
    of_ref[...] = v
    ob_ref[...] = v.astype(jnp.bfloat16)


def _prescale(table):
    return pl.pallas_call(
        _prescale_body,
        out_shape=[
            jax.ShapeDtypeStruct((_VOCAB, _D), jnp.float32),
            jax.ShapeDtypeStruct((_VOCAB, _D), jnp.bfloat16),
        ],
    )(table)


def _gather_body(tbl_hbm, idx_hbm, out_hbm, idx_v, buf0, buf1, gs0, gs1,
                 ss0, ss1):
    wid = lax.axis_index("s") * _NC + lax.axis_index("c")
    bpw = idx_hbm.shape[0] // _NW
    base = wid * bpw
    nch = bpw // _C
    bufs = (buf0, buf1)
    gsems = (gs0, gs1)
    ssems = (ss0, ss1)

    pltpu.sync_copy(idx_hbm.at[pl.ds(base, bpw)], idx_v)

    def gather_start(c, b):
        iv = idx_v[pl.ds(c * _C, _C)]
        pltpu.async_copy(tbl_hbm.at[iv], bufs[b], gsems[b])

    def gather_wait(b):
        iv = idx_v[pl.ds(0, _C)]
        pltpu.make_async_copy(tbl_hbm.at[iv], bufs[b], gsems[b]).wait()

    def scatter_start(c, b):
        pltpu.async_copy(bufs[b], out_hbm.at[pl.ds(base + c * _C, _C)],
                         ssems[b])

    def scatter_wait(b):
        pltpu.make_async_copy(bufs[b], out_hbm.at[pl.ds(base, _C)],
                              ssems[b]).wait()

    gather_start(0, 0)

    def pair(g, carry):
        for b in range(2):
            c = g + b
            nb = (b + 1) % 2
            @pl.when(c > 0)
            def _():
                scatter_wait(nb)

            @pl.when(c + 1 < nch)
            def _():
                gather_start(c + 1, nb)

            gather_wait(b)
            scatter_start(c, b)
        return carry

    lax.fori_loop(0, nch // 2, lambda i, cr: pair(i * 2, cr), 0)
    scatter_wait((nch - 1) % 2)


def _sc_gather(table_f32, idx_sc, n_total):
    mesh = plsc.VectorSubcoreMesh(core_axis_name="c", subcore_axis_name="s")
    bpw = idx_sc.shape[0] // _NW
    return pl.kernel(
        _gather_body,
        out_type=jax.ShapeDtypeStruct((n_total, _D), jnp.float32),
        mesh=mesh,
        scratch_types=[
            pltpu.VMEM((bpw,), jnp.int32),
            pltpu.VMEM((_C, _D), jnp.float32),
            pltpu.VMEM((_C, _D), jnp.float32),
            pltpu.SemaphoreType.DMA,
            pltpu.SemaphoreType.DMA,
            pltpu.SemaphoreType.DMA,
            pltpu.SemaphoreType.DMA,
        ],
    )(table_f32, idx_sc)


def _tc_body(part_ref, idx_ref, tbl_ref, o_ref):
    del part_ref
    idx = idx_ref[0, 0, :]
    ids = idx.reshape(_RB, 1)
    col = lax.broadcasted_iota(jnp.int32, (_RB, _VOCAB), 1)
    oh = (ids == col).astype(jnp.bfloat16)
    o_ref[...] = jnp.dot(oh, tbl_ref[...],
                         preferred_element_type=jnp.float32)


def _tc_fill(sc_out, tbl_bf, idx3, blk_off):
    n = sc_out.shape[0]
    nb_tc = idx3.shape[0] - blk_off
    return pl.pallas_call(
        _tc_body,
        grid=(nb_tc,),
        in_specs=[
            pl.BlockSpec(memory_space=pl.ANY),
            pl.BlockSpec((1, 1, _RB), lambda i: (i + blk_off, 0, 0)),
            pl.BlockSpec((_VOCAB, _D), lambda i: (0, 0)),
        ],
        out_specs=pl.BlockSpec((_RB, _D), lambda i: (i + blk_off, 0)),
        out_shape=jax.ShapeDtypeStruct((n, _D), jnp.float32),
        input_output_aliases={0: 0},
    )(sc_out, idx3, tbl_bf)


@jax.jit
def kernel(x, table):
    b, s = x.shape
    idx = x.reshape(-1).astype(jnp.int32)
    n = idx.shape[0]
    m_sc = (n * _SC_FRAC_NUM // _SC_FRAC_DEN) // (_NW * _C) * (_NW * _C)

    table_f32, table_bf = _prescale(table)
    sc_out = _sc_gather(table_f32, idx[:m_sc], n)
    idx3 = idx.reshape(n // _RB, 1, _RB)
    out = _tc_fill(sc_out, table_bf, idx3, m_sc // _RB)
    return out.reshape(b, s, _D)


# speedup vs baseline: 1.0730x; 1.0730x over previous
import math
import functools

import jax
import jax.numpy as jnp
from jax import lax
from jax.experimental import pallas as pl
from jax.experimental.pallas import tpu as pltpu
from jax.experimental.pallas import tpu_sc as plsc

_VOCAB = 256
_D = 2048
_SCALE = math.sqrt(_D)

_NC = 2
_NS = 16
_NW = _NC * _NS

_C = 16
_RB = 512

_SC_FRAC_NUM = 1
_SC_FRAC_DEN = 2


def _prescale_body(t_ref, of_ref, ob_ref):
    row = lax.broadcasted_iota(jnp.int32, (_VOCAB, _D), 0)
    v = jnp.where(row == 0, 0.0, t_ref[...] * _SCALE)
    of_ref[...] = v
    ob_ref[...] = v.astype(jnp.bfloat16)


def _prescale(table):
    return pl.pallas_call(
        _prescale_body,
        out_shape=[
            jax.ShapeDtypeStruct((_VOCAB, _D), jnp.float32),
            jax.ShapeDtypeStruct((_VOCAB, _D), jnp.bfloat16),
        ],
    )(table)


def _gather_body(tbl_hbm, idx_hbm, out_hbm, idx_v, buf0, buf1, gs0, gs1,
                 ss0, ss1):
    wid = lax.axis_index("s") * _NC + lax.axis_index("c")
    bpw = idx_hbm.shape[0] // _NW
    base = wid * bpw
    nch = bpw // _C
    bufs = (buf0, buf1)
    gsems = (gs0, gs1)
    ssems = (ss0, ss1)

    pltpu.sync_copy(idx_hbm.at[pl.ds(base, bpw)], idx_v)

    def gather_start(c, b):
        iv = idx_v[pl.ds(c * _C, _C)]
        pltpu.async_copy(tbl_hbm.at[iv], bufs[b], gsems[b])

    def gather_wait(b):
        iv = idx_v[pl.ds(0, _C)]
        pltpu.make_async_copy(tbl_hbm.at[iv], bufs[b], gsems[b]).wait()

    def scatter_start(c, b):
        pltpu.async_copy(bufs[b], out_hbm.at[pl.ds(base + c * _C, _C)],
                         ssems[b])

    def scatter_wait(b):
        pltpu.make_async_copy(bufs[b], out_hbm.at[pl.ds(base, _C)],
                              ssems[b]).wait()

    gather_start(0, 0)

    def pair(g, carry):
        for b in range(2):
            c = g + b
            nb = (b + 1) % 2
            @pl.when(c > 0)
            def _():
                scatter_wait(nb)

            @pl.when(c + 1 < nch)
            def _():
                gather_start(c + 1, nb)

            gather_wait(b)
            scatter_start(c, b)
        return carry

    lax.fori_loop(0, nch // 2, lambda i, cr: pair(i * 2, cr), 0)
    scatter_wait((nch - 1) % 2)


def _sc_gather(table_f32, idx_sc, n_total):
    mesh = plsc.VectorSubcoreMesh(core_axis_name="c", subcore_axis_name="s")
    bpw = idx_sc.shape[0] // _NW
    return pl.kernel(
        _gather_body,
        out_type=jax.ShapeDtypeStruct((n_total, _D), jnp.float32),
        mesh=mesh,
        scratch_types=[
            pltpu.VMEM((bpw,), jnp.int32),
            pltpu.VMEM((_C, _D), jnp.float32),
            pltpu.VMEM((_C, _D), jnp.float32),
            pltpu.SemaphoreType.DMA,
            pltpu.SemaphoreType.DMA,
            pltpu.SemaphoreType.DMA,
            pltpu.SemaphoreType.DMA,
        ],
    )(table_f32, idx_sc)


def _tc_body(part_ref, idx_ref, tbl_ref, o_ref):
    del part_ref
    idx = idx_ref[0, 0, :]
    ids = idx.reshape(_RB, 1)
    col = lax.broadcasted_iota(jnp.int32, (_RB, _VOCAB), 1)
    oh = (ids == col).astype(jnp.bfloat16)
    o_ref[...] = jnp.dot(oh, tbl_ref[...],
                         preferred_element_type=jnp.float32)


def _tc_fill(sc_out, tbl_bf, idx3, blk_off):
    n = sc_out.shape[0]
    nb_tc = idx3.shape[0] - blk_off
    return pl.pallas_call(
        _tc_body,
        grid=(nb_tc,),
        in_specs=[
            pl.BlockSpec(memory_space=pl.ANY),
            pl.BlockSpec((1, 1, _RB), lambda i: (i + blk_off, 0, 0)),
            pl.BlockSpec((_VOCAB, _D), lambda i: (0, 0)),
        ],
        out_specs=pl.BlockSpec((_RB, _D), lambda i: (i + blk_off, 0)),
        out_shape=jax.ShapeDtypeStruct((n, _D), jnp.float32),
        input_output_aliases={0: 0},
    )(sc_out, idx3, tbl_bf)


@jax.jit
def kernel(x, table):
    b, s = x.shape
    idx = x.reshape(-1).astype(jnp.int32)
    n = idx.shape[0]
    m_sc = (n * _SC_FRAC_NUM // _SC_FRAC_DEN) // (_NW * _C) * (_NW * _C)

    table_f32, table_bf = _prescale(table)
    sc_out = _sc_gather(table_f32, idx[:m_sc], n)
    idx3 = idx.reshape(n // _RB, 1, _RB)
    out = _tc_fill(sc_out, table_bf, idx3, m_sc // _RB)
    return out.reshape(b, s, _D)


# speedup vs baseline: 1.0783x; 1.0050x over previous
import math

import jax
import jax.numpy as jnp
from jax import lax
from jax.experimental import pallas as pl
from jax.experimental.pallas import tpu as pltpu
from jax.experimental.pallas import tpu_sc as plsc

_VOCAB = 256
_D = 2048
_SCALE = math.sqrt(_D)

_NC = 2
_NS = 16
_NW = _NC * _NS

_C = 16
_RB = 1024

_SC_FRAC_NUM = 1
_SC_FRAC_DEN = 2


def _prescale_body(t_ref, of_ref, ob_ref):
    row = lax.broadcasted_iota(jnp.int32, (_VOCAB, _D), 0)
    v = jnp.where(row == 0, 0.0, t_ref[...] * _SCALE)
    of_ref[...] = v
    ob_ref[...] = v.astype(jnp.bfloat16)


def _prescale(table):
    return pl.pallas_call(
        _prescale_body,
        out_shape=[
            jax.ShapeDtypeStruct((_VOCAB, _D), jnp.float32),
            jax.ShapeDtypeStruct((_VOCAB, _D), jnp.bfloat16),
        ],
    )(table)


def _gather_body(tbl_hbm, idx_hbm, out_hbm, idx_v, buf0, buf1, gs0, gs1,
                 ss0, ss1):
    wid = lax.axis_index("s") * _NC + lax.axis_index("c")
    bpw = idx_hbm.shape[0] // _NW
    base = wid * bpw
    nch = bpw // _C
    bufs = (buf0, buf1)
    gsems = (gs0, gs1)
    ssems = (ss0, ss1)

    pltpu.sync_copy(idx_hbm.at[pl.ds(base, bpw)], idx_v)

    def gather_start(c, b):
        iv = idx_v[pl.ds(c * _C, _C)]
        pltpu.async_copy(tbl_hbm.at[iv], bufs[b], gsems[b])

    def gather_wait(b):
        iv = idx_v[pl.ds(0, _C)]
        pltpu.make_async_copy(tbl_hbm.at[iv], bufs[b], gsems[b]).wait()

    def scatter_start(c, b):
        pltpu.async_copy(bufs[b], out_hbm.at[pl.ds(base + c * _C, _C)],
                         ssems[b])

    def scatter_wait(b):
        pltpu.make_async_copy(bufs[b], out_hbm.at[pl.ds(base, _C)],
                              ssems[b]).wait()

    gather_start(0, 0)

    def pair(g, carry):
        for b in range(2):
            c = g + b
            nb = (b + 1) % 2
            @pl.when(c > 0)
            def _():
                scatter_wait(nb)

            @pl.when(c + 1 < nch)
            def _():
                gather_start(c + 1, nb)

            gather_wait(b)
            scatter_start(c, b)
        return carry

    lax.fori_loop(0, nch // 2, lambda i, cr: pair(i * 2, cr), 0)
    scatter_wait((nch - 1) % 2)


def _sc_gather(table_f32, idx_sc, n_total):
    mesh = plsc.VectorSubcoreMesh(core_axis_name="c", subcore_axis_name="s")
    bpw = idx_sc.shape[0] // _NW
    return pl.kernel(
        _gather_body,
        out_type=jax.ShapeDtypeStruct((n_total, _D), jnp.float32),
        mesh=mesh,
        scratch_types=[
            pltpu.VMEM((bpw,), jnp.int32),
            pltpu.VMEM((_C, _D), jnp.float32),
            pltpu.VMEM((_C, _D), jnp.float32),
            pltpu.SemaphoreType.DMA,
            pltpu.SemaphoreType.DMA,
            pltpu.SemaphoreType.DMA,
            pltpu.SemaphoreType.DMA,
        ],
    )(table_f32, idx_sc)


def _tc_body(part_ref, idx_ref, tbl_ref, o_ref):
    del part_ref
    idx = idx_ref[0, 0, :]
    ids = idx.reshape(_RB, 1)
    col = lax.broadcasted_iota(jnp.int32, (_RB, _VOCAB), 1)
    oh = (ids == col).astype(jnp.bfloat16)
    o_ref[...] = jnp.dot(oh, tbl_ref[...],
                         preferred_element_type=jnp.float32)


def _tc_fill(sc_out, tbl_bf, idx3, blk_off):
    n = sc_out.shape[0]
    nb_tc = idx3.shape[0] - blk_off
    return pl.pallas_call(
        _tc_body,
        grid=(nb_tc,),
        in_specs=[
            pl.BlockSpec(memory_space=pl.ANY),
            pl.BlockSpec((1, 1, _RB), lambda i: (i + blk_off, 0, 0)),
            pl.BlockSpec((_VOCAB, _D), lambda i: (0, 0)),
        ],
        out_specs=pl.BlockSpec((_RB, _D), lambda i: (i + blk_off, 0)),
        out_shape=jax.ShapeDtypeStruct((n, _D), jnp.float32),
        input_output_aliases={0: 0},
    )(sc_out, idx3, tbl_bf)


@jax.jit
def kernel(x, table):
    b, s = x.shape
    idx = x.reshape(-1).astype(jnp.int32)
    n = idx.shape[0]
    m_sc = (n * _SC_FRAC_NUM // _SC_FRAC_DEN) // (_NW * _C) * (_NW * _C)

    table_f32, table_bf = _prescale(table)
    sc_out = _sc_gather(table_f32, idx[:m_sc], n)
    idx3 = idx.reshape(n // _RB, 1, _RB)
    out = _tc_fill(sc_out, table_bf, idx3, m_sc // _RB)
    return out.reshape(b, s, _D)
